# baseline (speedup 1.0000x reference)
import jax
import jax.numpy as jnp
from jax import lax
from jax.experimental import pallas as pl
from jax.experimental.pallas import tpu as pltpu

H = 32
W = 32
F = 384
HW = H * W


def _pos_body(row_ref, col_ref, out_hbm, scratch, sem):
    col_t = col_ref[...].T
    row_t = row_ref[...].T
    lane = lax.broadcasted_iota(jnp.int32, (W, HW), 1)
    sub = lax.broadcasted_iota(jnp.int32, (W, HW), 0)
    tile_mask = (lane % W == sub).astype(jnp.float32)
    rep_mask = (lane // W == sub).astype(jnp.float32)
    scratch[:F] = jnp.dot(col_t, tile_mask, precision=lax.Precision.HIGHEST,
                          preferred_element_type=jnp.float32)
    scratch[F:] = jnp.dot(row_t, rep_mask, precision=lax.Precision.HIGHEST,
                          preferred_element_type=jnp.float32)
    b = out_hbm.shape[0]
    copies = [
        pltpu.make_async_copy(scratch, out_hbm.at[i], sem) for i in range(b)
    ]
    for c in copies:
        c.start()
    for c in copies:
        c.wait()


def kernel(x, row_embed, col_embed):
    b = x.shape[0]
    out = pl.pallas_call(
        _pos_body,
        in_specs=[
            pl.BlockSpec((H, F), lambda: (0, 0)),
            pl.BlockSpec((W, F), lambda: (0, 0)),
        ],
        out_specs=pl.BlockSpec(memory_space=pl.ANY),
        out_shape=jax.ShapeDtypeStruct((b, 2 * F, HW), jnp.float32),
        scratch_shapes=[
            pltpu.VMEM((2 * F, HW), jnp.float32),
            pltpu.SemaphoreType.DMA,
        ],
    )(row_embed, col_embed)
    return out.reshape(b, 2 * F, H, W)


# speedup vs baseline: 1.0142x; 1.0142x over previous
import jax
import jax.numpy as jnp
from jax import lax
from jax.experimental import pallas as pl
from jax.experimental.pallas import tpu as pltpu

H = 32
W = 32
F = 384
HW = H * W


def _pos_body(row_ref, col_ref, out_hbm, scratch, sems):
    col_t = col_ref[...].T
    row_t = row_ref[...].T
    lane = lax.broadcasted_iota(jnp.int32, (W, HW), 1)
    sub = lax.broadcasted_iota(jnp.int32, (W, HW), 0)
    tile_mask = (lane % W == sub).astype(jnp.float32)
    rep_mask = (lane // W == sub).astype(jnp.float32)
    scratch[:F] = jnp.dot(col_t, tile_mask, precision=lax.Precision.HIGHEST,
                          preferred_element_type=jnp.float32)
    scratch[F:] = jnp.dot(row_t, rep_mask, precision=lax.Precision.HIGHEST,
                          preferred_element_type=jnp.float32)
    b = out_hbm.shape[0]
    copies = [
        pltpu.make_async_copy(scratch, out_hbm.at[i], sems.at[i])
        for i in range(b)
    ]
    for c in copies:
        c.start()
    for c in copies:
        c.wait()


def kernel(x, row_embed, col_embed):
    b = x.shape[0]
    out = pl.pallas_call(
        _pos_body,
        in_specs=[
            pl.BlockSpec((H, F), lambda: (0, 0)),
            pl.BlockSpec((W, F), lambda: (0, 0)),
        ],
        out_specs=pl.BlockSpec(memory_space=pl.ANY),
        out_shape=jax.ShapeDtypeStruct((b, 2 * F, HW), jnp.float32),
        scratch_shapes=[
            pltpu.VMEM((2 * F, HW), jnp.float32),
            pltpu.SemaphoreType.DMA((b,)),
        ],
    )(row_embed, col_embed)
    return out.reshape(b, 2 * F, H, W)
